# Initial kernel scaffold; baseline (speedup 1.0000x reference)
import jax
import jax.numpy as jnp
from jax.experimental import pallas as pl


def kernel(coords, W1, b1, W2, b2, connectivity, prop_E, prop_A, prop_I22, elem_lengths, elem_directions, elem_load, bc_disp, bc_rot):
    raise NotImplementedError("write your pallas kernel here")


# speedup vs baseline: 19.1053x; 19.1053x over previous
import jax
import jax.numpy as jnp
from jax import lax
from jax.experimental import pallas as pl
from jax.experimental.pallas import tpu as pltpu
from jax.experimental.pallas import tpu_sc as plsc

NN = 100000
NE = 1600000
CH = 128
KB = CH // 128
NCHUNK = NE // CH
NC = 2
NS = 16
NW = NC * NS
TW = 16
TBW = 128
VB = CH // 16
AR = NN // 16

_F32 = jnp.float32
_I32 = jnp.int32


def _mlp_body(coords_ref, W1_ref, b1_ref, W2_ref, b2_ref, pred_ref, tab_ref):
    c = coords_ref[...]
    W1 = W1_ref[...]
    W2 = W2_ref[...]
    z = jnp.dot(c, W1, preferred_element_type=_F32) + b1_ref[...][None, :]
    h = jnp.tanh(z)
    pred = jnp.dot(h, W2, preferred_element_type=_F32) + b2_ref[...][None, :]
    pred_ref[...] = pred
    t = 1.0 - h * h
    dims = (((1,), (1,)), ((), ()))
    g0 = lax.dot_general(t * W2[:, 0][None, :], W1, dims,
                         preferred_element_type=_F32)
    g1 = lax.dot_general(t * W2[:, 1][None, :], W1, dims,
                         preferred_element_type=_F32)
    g2 = lax.dot_general(t * W2[:, 2][None, :], W1, dims,
                         preferred_element_type=_F32)
    pad = jnp.zeros((c.shape[0], TBW - 12), _F32)
    tab_ref[...] = jnp.concatenate([pred, g0, g1, g2, pad], axis=1)


def _mlp_call(coords, W1, b1, W2, b2):
    B = 5000
    grid = NN // B
    return pl.pallas_call(
        _mlp_body,
        grid=(grid,),
        in_specs=[
            pl.BlockSpec((B, 3), lambda i: (i, 0)),
            pl.BlockSpec((3, 64), lambda i: (0, 0)),
            pl.BlockSpec((64,), lambda i: (0,)),
            pl.BlockSpec((64, 3), lambda i: (0, 0)),
            pl.BlockSpec((3,), lambda i: (0,)),
        ],
        out_specs=[
            pl.BlockSpec((B, 3), lambda i: (i, 0)),
            pl.BlockSpec((B, TBW), lambda i: (i, 0)),
        ],
        out_shape=[
            jax.ShapeDtypeStruct((NN, 3), _F32),
            jax.ShapeDtypeStruct((NN, TBW), _F32),
        ],
    )(coords, W1, b1, W2, b2)


def _rsqrt(n2):
    y = 1.0 / (0.1237 + 0.8763 * n2)
    for _ in range(5):
        y = y * (1.5 - 0.5 * n2 * y * y)
    return y


def _rsqrt_unit(n2):
    y = 1.5 - 0.5 * n2
    y = y * (1.5 - 0.5 * n2 * y * y)
    return y


def _sc_body(ntab, idx_i, idx_j, dirs, loads, Lh, peh, pah, pih,
             out_accF, out_accE, out_part,
             idx_i_v, idx_j_v, idx_i_d, idx_j_d,
             dirs_v, loads_v, L_v, pe_v, pa_v, pi_v,
             rows_i, rows_j, out_i, out_j, part_v,
             acc_spm, sem_g, sem_s):
    cidx = lax.axis_index("c")
    sidx = lax.axis_index("s")
    wid = sidx * NC + cidx
    acc_sh = acc_spm

    NBLK = AR // CH
    TAIL = AR - NBLK * CH
    cnt = (NBLK + NS - 1 - sidx) // NS

    def _zero_out_i():
        z16 = jnp.zeros((16,), _F32)
        for r in range(CH):
            for c in range(8):
                out_i[r, pl.ds(c * 16, 16)] = z16

    def _zero_acc():
        def _z(t, _):
            pltpu.sync_copy(out_i,
                            acc_sh.at[pl.ds((sidx + t * NS) * CH, CH)])
            return 0
        lax.fori_loop(0, cnt, _z, 0)

        @pl.when(sidx == 0)
        def _():
            pltpu.sync_copy(out_i.at[pl.ds(0, TAIL)],
                            acc_sh.at[pl.ds(NBLK * CH, TAIL)])

    def _publish(dst):
        def _p(t, _):
            blk = pl.ds((sidx + t * NS) * CH, CH)
            pltpu.sync_copy(acc_sh.at[blk], dst.at[blk])
            return 0
        lax.fori_loop(0, cnt, _p, 0)

        @pl.when(sidx == 0)
        def _():
            tail = pl.ds(NBLK * CH, TAIL)
            pltpu.sync_copy(acc_sh.at[tail], dst.at[tail])

    _zero_out_i()
    _zero_acc()
    plsc.subcore_barrier()

    iota16 = lax.iota(_I32, 16)

    tr_masks = [(iota16 & o) != 0 for o in (8, 4, 2, 1)]
    tr_perms = [iota16 ^ o for o in (8, 4, 2, 1)]
    _gdn = lax.GatherDimensionNumbers(
        offset_dims=(), collapsed_slice_dims=(0,), start_index_map=(0,))

    def _perm(v, p):
        return lax.gather(v, p[:, None], _gdn, (1,),
                          mode=lax.GatherScatterMode.PROMISE_IN_BOUNDS)

    def _transpose16(vs):
        for o, msk, prm in zip((8, 4, 2, 1), tr_masks, tr_perms):
            new = list(vs)
            for i in range(16):
                if i & o == 0:
                    new[i] = jnp.where(msk, _perm(vs[i + o], prm), vs[i])
                else:
                    new[i] = jnp.where(msk, vs[i], _perm(vs[i - o], prm))
            vs = new
        return vs

    def vec_body(k, carry):
        kin, seps, skap, qmx, lmx, lsm = carry
        b = k * 16

        xx = dirs_v[0, pl.ds(b, 16)]
        xy = dirs_v[1, pl.ds(b, 16)]
        xz = dirs_v[2, pl.ds(b, 16)]
        qx = loads_v[0, pl.ds(b, 16)]
        qy = loads_v[1, pl.ds(b, 16)]
        qz = loads_v[2, pl.ds(b, 16)]
        Le = L_v[pl.ds(b, 16)]
        pe = pe_v[pl.ds(b, 16)]
        pa = pa_v[pl.ds(b, 16)]
        pi2 = pi_v[pl.ds(b, 16)]

        ti = _transpose16([rows_i[b + l, pl.ds(0, 16)] for l in range(16)])
        (uxi, uzi, phi, gxi0, gxi1, gxi2, gzi0, gzi1, gzi2,
         gpi0, gpi1, gpi2) = ti[:12]
        tj = _transpose16([rows_j[b + l, pl.ds(0, 16)] for l in range(16)])
        (uxj, uzj, phj, gxj0, gxj1, gxj2, gzj0, gzj1, gzj2,
         gpj0, gpj1, gpj2) = tj[:12]

        EA = pe * pa
        EI = pe * pi2
        inv_L = 1.0 / Le
        dux = uxj - uxi
        duz = uzj - uzi
        du_ax = dux * xx + duz * xz
        eps_fd = du_ax * inv_L
        N_fd = EA * eps_fd

        par = jnp.abs(xy) > 0.99
        zrx = jnp.where(par, xy, -xz)
        zry = jnp.where(par, -xx, jnp.float32(0.0))
        zrz = jnp.where(par, jnp.float32(0.0), xx)
        rz = _rsqrt(zrx * zrx + zry * zry + zrz * zrz)
        zhx = zrx * rz; zhy = zry * rz; zhz = zrz * rz
        yrx = zhy * xz - zhz * xy
        yry = zhz * xx - zhx * xz
        yrz = zhx * xy - zhy * xx
        ry = _rsqrt_unit(yrx * yrx + yry * yry + yrz * yrz)
        yhx = yrx * ry; yhy = yry * ry; yhz = yrz * ry

        du_tr = dux * zhx + duz * zhz
        kap_fd = (phj - phi) * inv_L
        invL2 = inv_L * inv_L
        EIL = EI * inv_L
        sphi = phi + phj
        V_fd = 12.0 * EIL * invL2 * du_tr - 6.0 * EI * invL2 * sphi
        M_yi = 6.0 * EI * invL2 * du_tr - EIL * (4.0 * phi + 2.0 * phj)
        M_yj = 6.0 * EI * invL2 * du_tr - EIL * (2.0 * phi + 4.0 * phj)

        gxi_d = gxi0 * xx + gxi1 * xy + gxi2 * xz
        gzi_d = gzi0 * xx + gzi1 * xy + gzi2 * xz
        gxj_d = gxj0 * xx + gxj1 * xy + gxj2 * xz
        gzj_d = gzj0 * xx + gzj1 * xy + gzj2 * xz
        eps_ag = 0.5 * ((xx * gxi_d + xz * gzi_d) + (xx * gxj_d + xz * gzj_d))
        kap_ag = 0.5 * ((gpi0 * xx + gpi1 * xy + gpi2 * xz)
                        + (gpj0 * xx + gpj1 * xy + gpj2 * xz))

        r_kin = 0.5 * sphi - du_tr * inv_L
        kin = kin + r_kin * r_kin
        de = eps_ag - eps_fd
        seps = seps + de * de
        dk = kap_ag - kap_fd
        skap = skap + dk * dk
        qmx = jnp.maximum(qmx, jnp.maximum(jnp.abs(qx),
                                           jnp.maximum(jnp.abs(qy),
                                                       jnp.abs(qz))))
        lmx = jnp.maximum(lmx, Le)
        lsm = lsm + Le

        Fx = N_fd * xx + V_fd * zhx
        Fy = N_fd * xy + V_fd * zhy
        Fz = N_fd * xz + V_fd * zhz
        hl = Le * 0.5
        ex = qx * hl; ey = qy * hl; ez = qz * hl

        z16v = jnp.zeros((16,), _F32)
        oi = _transpose16([Fx + ex, Fy + ey, Fz + ez,
                           M_yi * yhx, M_yi * yhy, M_yi * yhz,
                           z16v, z16v, z16v,
                           z16v, z16v, z16v, z16v, z16v, z16v, z16v])
        oj = _transpose16([ex - Fx, ey - Fy, ez - Fz,
                           M_yj * yhx, M_yj * yhy, M_yj * yhz,
                           z16v, z16v, z16v,
                           z16v, z16v, z16v, z16v, z16v, z16v, z16v])
        niv = (idx_i_v[0, pl.ds(b, 16)] & 15) * 8
        njv = (idx_j_v[0, pl.ds(b, 16)] & 15) * 8
        for l in range(16):
            r = b + l
            for c in range(8):
                out_i[r, pl.ds(c * 16, 16)] = z16v
                out_j[r, pl.ds(c * 16, 16)] = z16v
            hi_i = niv[l] == 120
            hi_j = njv[l] == 120
            vi = jnp.where(hi_i, _perm(oi[l], tr_perms[0]), oi[l])
            vj = jnp.where(hi_j, _perm(oj[l], tr_perms[0]), oj[l])
            out_i[r, pl.ds(jnp.minimum(niv[l], 112), 16)] = vi
            out_j[r, pl.ds(jnp.minimum(njv[l], 112), 16)] = vj
        return kin, seps, skap, qmx, lmx, lsm

    def chunk_body(t, carry):
        cid = wid + t * NW
        pltpu.sync_copy(idx_i.at[cid], idx_i_v)
        pltpu.sync_copy(idx_j.at[cid], idx_j_v)
        for v in range(VB):
            nv_i = idx_i_v[0, pl.ds(v * 16, 16)]
            nv_j = idx_j_v[0, pl.ds(v * 16, 16)]
            idx_i_d[0, pl.ds(v * 16, 16)] = lax.shift_right_logical(nv_i, 4)
            idx_j_d[0, pl.ds(v * 16, 16)] = lax.shift_right_logical(nv_j, 4)
        pltpu.sync_copy(dirs.at[cid], dirs_v)
        pltpu.sync_copy(loads.at[cid], loads_v)
        pltpu.sync_copy(Lh.at[cid], L_v)
        pltpu.sync_copy(peh.at[cid], pe_v)
        pltpu.sync_copy(pah.at[cid], pa_v)
        pltpu.sync_copy(pih.at[cid], pi_v)
        cps = []
        for k in range(KB):
            cps.append(pltpu.async_copy(
                ntab.at[idx_i_v.at[k]],
                rows_i.at[pl.ds(k * 128, 128)], sem_g))
            cps.append(pltpu.async_copy(
                ntab.at[idx_j_v.at[k]],
                rows_j.at[pl.ds(k * 128, 128)], sem_g))
        for cp in cps:
            cp.wait()
        carry = lax.fori_loop(0, VB, vec_body, carry)
        scps = []
        for k in range(KB):
            scps.append(pltpu.async_copy(
                out_i.at[pl.ds(k * 128, 128)], acc_sh.at[idx_i_d.at[k]],
                sem_s, add=True))
            scps.append(pltpu.async_copy(
                out_j.at[pl.ds(k * 128, 128)], acc_sh.at[idx_j_d.at[k]],
                sem_s, add=True))
        for cp in scps:
            cp.wait()
        return carry

    nfull = NCHUNK // NW
    count = nfull + jnp.where(wid < NCHUNK - nfull * NW, 1, 0)
    zv = jnp.zeros((16,), _F32)
    kin, seps, skap, qmx, lmx, lsm = lax.fori_loop(
        0, count, chunk_body, (zv, zv, zv, zv, zv, zv))

    part_v[0] = kin
    part_v[1] = seps
    part_v[2] = skap
    part_v[3] = qmx
    part_v[4] = lmx
    part_v[5] = lsm
    pltpu.sync_copy(part_v, out_part.at[wid])

    plsc.subcore_barrier()
    _publish(out_accF.at[cidx])
    plsc.subcore_barrier()
    _zero_out_i()
    _zero_acc()
    plsc.subcore_barrier()

    def ext_vec_body(k, _):
        b = k * 16
        qx = loads_v[0, pl.ds(b, 16)]
        qy = loads_v[1, pl.ds(b, 16)]
        qz = loads_v[2, pl.ds(b, 16)]
        hl = L_v[pl.ds(b, 16)] * 0.5
        z16v = jnp.zeros((16,), _F32)
        oe = _transpose16([qx * hl, qy * hl, qz * hl,
                           z16v, z16v, z16v, z16v, z16v,
                           z16v, z16v, z16v, z16v, z16v, z16v, z16v, z16v])
        niv = (idx_i_v[0, pl.ds(b, 16)] & 15) * 8
        njv = (idx_j_v[0, pl.ds(b, 16)] & 15) * 8
        for l in range(16):
            r = b + l
            for c in range(8):
                out_i[r, pl.ds(c * 16, 16)] = z16v
                out_j[r, pl.ds(c * 16, 16)] = z16v
            vi = jnp.where(niv[l] == 120, _perm(oe[l], tr_perms[0]), oe[l])
            vj = jnp.where(njv[l] == 120, _perm(oe[l], tr_perms[0]), oe[l])
            out_i[r, pl.ds(jnp.minimum(niv[l], 112), 16)] = vi
            out_j[r, pl.ds(jnp.minimum(njv[l], 112), 16)] = vj
        return 0

    def ext_chunk_body(t, _):
        cid = wid + t * NW
        pltpu.sync_copy(idx_i.at[cid], idx_i_v)
        pltpu.sync_copy(idx_j.at[cid], idx_j_v)
        for v in range(VB):
            nv_i = idx_i_v[0, pl.ds(v * 16, 16)]
            nv_j = idx_j_v[0, pl.ds(v * 16, 16)]
            idx_i_d[0, pl.ds(v * 16, 16)] = lax.shift_right_logical(nv_i, 4)
            idx_j_d[0, pl.ds(v * 16, 16)] = lax.shift_right_logical(nv_j, 4)
        pltpu.sync_copy(loads.at[cid], loads_v)
        pltpu.sync_copy(Lh.at[cid], L_v)
        lax.fori_loop(0, VB, ext_vec_body, 0)
        scps = []
        for k in range(KB):
            scps.append(pltpu.async_copy(
                out_i.at[pl.ds(k * 128, 128)], acc_sh.at[idx_i_d.at[k]],
                sem_s, add=True))
            scps.append(pltpu.async_copy(
                out_j.at[pl.ds(k * 128, 128)], acc_sh.at[idx_j_d.at[k]],
                sem_s, add=True))
        for cp in scps:
            cp.wait()
        return 0

    lax.fori_loop(0, count, ext_chunk_body, 0)
    plsc.subcore_barrier()
    _publish(out_accE.at[cidx])


def _sc_call(ntab, idx_i, idx_j, dirs, loads, Lh, peh, pah, pih):
    mesh = plsc.VectorSubcoreMesh(core_axis_name="c", subcore_axis_name="s")
    f = pl.kernel(
        _sc_body,
        out_type=[
            jax.ShapeDtypeStruct((NC, AR, TBW), _F32),
            jax.ShapeDtypeStruct((NC, AR, TBW), _F32),
            jax.ShapeDtypeStruct((NW, 6, 16), _F32),
        ],
        mesh=mesh,
        scratch_types=[
            pltpu.VMEM((KB, 128), _I32),
            pltpu.VMEM((KB, 128), _I32),
            pltpu.VMEM((KB, 128), _I32),
            pltpu.VMEM((KB, 128), _I32),
            pltpu.VMEM((3, CH), _F32),
            pltpu.VMEM((3, CH), _F32),
            pltpu.VMEM((CH,), _F32),
            pltpu.VMEM((CH,), _F32),
            pltpu.VMEM((CH,), _F32),
            pltpu.VMEM((CH,), _F32),
            pltpu.VMEM((CH, TBW), _F32),
            pltpu.VMEM((CH, TBW), _F32),
            pltpu.VMEM((CH, TBW), _F32),
            pltpu.VMEM((CH, TBW), _F32),
            pltpu.VMEM((6, 16), _F32),
            pltpu.VMEM_SHARED((AR, TBW), _F32),
            pltpu.SemaphoreType.DMA,
            pltpu.SemaphoreType.DMA,
        ],
    )
    return f(ntab, idx_i, idx_j, dirs, loads, Lh, peh, pah, pih)


def _red_body(accF_ref, accE_ref, bd_ref, br_ref, part_ref, out_ref):
    i = pl.program_id(0)
    a = accF_ref[0] + accF_ref[1]
    e = accE_ref[0] + accE_ref[1]
    F_all = a[:, 0:3]
    M = a[:, 3:6]
    Fext = e[:, 0:3]
    bd = bd_ref[...][:, 0]
    br = br_ref[...][:, 0]
    fd = bd < 0.5
    fr = br < 0.5
    pin = (bd > 0.5) & fr
    zero3 = jnp.zeros_like(F_all)
    s_fall2 = jnp.sum(jnp.where(fd[:, None], F_all * F_all, zero3))
    s_fext2 = jnp.sum(jnp.where(fd[:, None], Fext * Fext, zero3))
    s_m2r = jnp.sum(jnp.where(fr[:, None], M * M, zero3))
    s_m2p = jnp.sum(jnp.where(pin[:, None], M * M, zero3))
    cd = jnp.sum(fd.astype(_F32))
    cr = jnp.sum(fr.astype(_F32))
    cp = jnp.sum(pin.astype(_F32))

    ri = lax.broadcasted_iota(_I32, (8, 128), 0)
    ci = lax.broadcasted_iota(_I32, (8, 128), 1)

    def put(b, r, c, v):
        return b + jnp.where((ri == r) & (ci == c), v, jnp.float32(0.0))

    blk = jnp.zeros((8, 128), _F32)
    blk = put(blk, 0, 0, s_fall2)
    blk = put(blk, 0, 1, s_fext2)
    blk = put(blk, 0, 2, s_m2r)
    blk = put(blk, 0, 3, s_m2p)
    blk = put(blk, 0, 4, cd)
    blk = put(blk, 0, 5, cr)
    blk = put(blk, 0, 6, cp)

    @pl.when(i == 0)
    def _():
        p = part_ref[...]
        e = jnp.zeros((8, 128), _F32)
        e = put(e, 1, 0, jnp.sum(p[:, 0, :]))
        e = put(e, 1, 1, jnp.sum(p[:, 1, :]))
        e = put(e, 1, 2, jnp.sum(p[:, 2, :]))
        e = put(e, 1, 3, jnp.max(p[:, 3, :]))
        e = put(e, 1, 4, jnp.max(p[:, 4, :]))
        e = put(e, 1, 5, jnp.sum(p[:, 5, :]))
        out_ref[...] = e

    out_ref[...] = out_ref[...] + blk


def _red_call(accF, accE, bc_disp, bc_rot, parts):
    B = 2000
    grid = NN // B
    return pl.pallas_call(
        _red_body,
        grid=(grid,),
        in_specs=[
            pl.BlockSpec((NC, B, 8), lambda i: (0, i, 0)),
            pl.BlockSpec((NC, B, 8), lambda i: (0, i, 0)),
            pl.BlockSpec((B, 1), lambda i: (i, 0)),
            pl.BlockSpec((B, 1), lambda i: (i, 0)),
            pl.BlockSpec((NW, 6, 16), lambda i: (0, 0, 0)),
        ],
        out_specs=pl.BlockSpec((8, 128), lambda i: (0, 0)),
        out_shape=jax.ShapeDtypeStruct((8, 128), _F32),
    )(accF, accE, bc_disp, bc_rot, parts)


def kernel(coords, W1, b1, W2, b2, connectivity, prop_E, prop_A, prop_I22,
           elem_lengths, elem_directions, elem_load, bc_disp, bc_rot):
    pred, ntab = _mlp_call(coords, W1, b1, W2, b2)

    idx_i = connectivity[:, 0].reshape(NCHUNK, KB, 128)
    idx_j = connectivity[:, 1].reshape(NCHUNK, KB, 128)
    dirs = elem_directions.reshape(NCHUNK, CH, 3).transpose(0, 2, 1)
    loads = elem_load.reshape(NCHUNK, CH, 3).transpose(0, 2, 1)
    Lh = elem_lengths.reshape(NCHUNK, CH)
    peh = prop_E.reshape(NCHUNK, CH)
    pah = prop_A.reshape(NCHUNK, CH)
    pih = prop_I22.reshape(NCHUNK, CH)

    accF8, accE8, parts = _sc_call(ntab, idx_i, idx_j, dirs, loads,
                                   Lh, peh, pah, pih)
    accF = accF8.reshape(NC, NN, 8)
    accE = accE8.reshape(NC, NN, 8)
    S = _red_call(accF, accE, bc_disp, bc_rot, parts)

    s_fall2 = S[0, 0]
    s_fext2 = S[0, 1]
    s_m2r = S[0, 2]
    s_m2p = S[0, 3]
    cd = S[0, 4]
    cr = S[0, 5]
    cpin = S[0, 6]
    s_kin = S[1, 0]
    s_eps = S[1, 1]
    s_kap = S[1, 2]
    qmax = S[1, 3]
    lmax = S[1, 4]
    lsum = S[1, 5]

    F_char = jnp.clip(jnp.sqrt(s_fext2 / (3.0 * cd)), 1.0, None)
    q_max = jnp.clip(qmax, 1.0, None)
    M_char = jnp.clip(q_max * lmax * lsum / 8.0, 1.0, None)
    L_force = s_fall2 / (F_char * F_char) / (3.0 * cd)
    L_moment = s_m2r / (M_char * M_char) / (3.0 * cr)
    L_neumann = jnp.where(cpin > 0,
                          s_m2p / (M_char * M_char) / jnp.maximum(3.0 * cpin, 1.0),
                          jnp.float32(0.0))
    L_kin = s_kin / NE
    L_consist = (s_eps + s_kap) / NE
    total = (L_force + L_moment + L_neumann + 0.1 * L_kin + L_consist)
    return total.astype(_F32), pred
